# baseline (speedup 1.0000x reference)
import functools

import jax
import jax.numpy as jnp
from jax import lax
from jax.experimental import pallas as pl
from jax.experimental.pallas import tpu as pltpu
from jax.experimental.pallas import tpu_sc as plsc

N_NODES = 10000
D_FEAT = 128
N_EDGES = 320000

NC = 2
NS = 16
NW = NC * NS

CHUNK = 80
N_CHUNKS = N_EDGES // CHUNK
T_CHUNKS = N_CHUNKS // NW

ACC_ROWS = 10112
ROWS_PER_TILE = ACC_ROWS // NS

NBUF = 4


def _sc_partial_sums(x, src, dst, zeros):
  mesh = plsc.VectorSubcoreMesh(core_axis_name="c", subcore_axis_name="s")

  @functools.partial(
      pl.kernel,
      mesh=mesh,
      out_type=jax.ShapeDtypeStruct((NC, ACC_ROWS, D_FEAT), jnp.float32),
      scratch_types=[
          *[pltpu.VMEM((CHUNK,), jnp.int32) for _ in range(NBUF)],
          *[pltpu.VMEM((CHUNK,), jnp.int32) for _ in range(NBUF)],
          *[pltpu.VMEM((CHUNK, D_FEAT), jnp.float32) for _ in range(NBUF)],
          pltpu.VMEM_SHARED((ACC_ROWS, D_FEAT), jnp.float32),
          *[pltpu.SemaphoreType.DMA for _ in range(2 * NBUF)],
      ],
  )
  def k(x_hbm, src_hbm, dst_hbm, zeros_hbm, out_hbm, *refs):
    sidx = refs[0:NBUF]
    didx = refs[NBUF:2 * NBUF]
    bufs = refs[2 * NBUF:3 * NBUF]
    acc = refs[3 * NBUF]
    gsems = refs[3 * NBUF + 1:3 * NBUF + 1 + NBUF]
    isems = refs[3 * NBUF + 1 + NBUF:3 * NBUF + 1 + 2 * NBUF]

    c = lax.axis_index("c")
    s = lax.axis_index("s")
    wid = s * NC + c
    base_chunk = wid * T_CHUNKS

    pltpu.sync_copy(zeros_hbm, acc.at[pl.ds(s * ROWS_PER_TILE, ROWS_PER_TILE)])
    plsc.subcore_barrier()

    def idx_load(chunk, p):
      e = chunk * CHUNK
      return (
          pltpu.make_async_copy(src_hbm.at[pl.ds(e, CHUNK)], sidx[p], isems[p]),
          pltpu.make_async_copy(dst_hbm.at[pl.ds(e, CHUNK)], didx[p], isems[p]),
      )

    def idx_start(chunk, p):
      a, b = idx_load(chunk, p)
      a.start()
      b.start()

    def idx_wait(chunk, p):
      a, b = idx_load(chunk, p)
      a.wait()
      b.wait()

    def gather(p):
      return pltpu.make_async_copy(x_hbm.at[sidx[p]], bufs[p], gsems[p])

    def scatter_add(p):
      pltpu.sync_copy(bufs[p], acc.at[didx[p]], add=True)

    for p in range(NBUF):
      idx_start(base_chunk + p, p)
    for p in range(NBUF - 1):
      idx_wait(base_chunk + p, p)
      gather(p).start()

    def chunk_step(t, p):
      p_new = (p + NBUF - 1) % NBUF

      @pl.when(t + NBUF - 1 < T_CHUNKS)
      def _():
        idx_wait(base_chunk + t + NBUF - 1, p_new)
        gather(p_new).start()

      gather(p).wait()
      scatter_add(p)

      @pl.when(t + NBUF < T_CHUNKS)
      def _():
        idx_start(base_chunk + t + NBUF, p)

    def body(i, carry):
      t0 = NBUF * i
      for p in range(NBUF):
        chunk_step(t0 + p, p)
      return carry

    n_main = T_CHUNKS // NBUF
    lax.fori_loop(0, n_main, body, 0)
    chunk_step(T_CHUNKS - 1, (T_CHUNKS - 1) % NBUF)

    plsc.subcore_barrier()

    pltpu.sync_copy(
        acc.at[pl.ds(s * ROWS_PER_TILE, ROWS_PER_TILE)],
        out_hbm.at[c, pl.ds(s * ROWS_PER_TILE, ROWS_PER_TILE)],
    )

  return k(x, src, dst, zeros)


def _combine_body(a_ref, b_ref, o_ref):
  o_ref[...] = a_ref[0] + b_ref[0]


_BLK = 1000


def _combine(partials):
  return pl.pallas_call(
      _combine_body,
      grid=(N_NODES // _BLK,),
      in_specs=[
          pl.BlockSpec((1, _BLK, D_FEAT), lambda i: (0, i, 0)),
          pl.BlockSpec((1, _BLK, D_FEAT), lambda i: (1, i, 0)),
      ],
      out_specs=pl.BlockSpec((_BLK, D_FEAT), lambda i: (i, 0)),
      out_shape=jax.ShapeDtypeStruct((N_NODES, D_FEAT), jnp.float32),
  )(partials, partials)


def kernel(X, edge_index):
  src = edge_index[1]
  dst = edge_index[0]
  zeros = jnp.zeros((ROWS_PER_TILE, D_FEAT), jnp.float32)
  partials = _sc_partial_sums(X, src, dst, zeros)
  return _combine(partials)


# speedup vs baseline: 1.2214x; 1.2214x over previous
import functools

import jax
import jax.numpy as jnp
from jax import lax
from jax.experimental import pallas as pl
from jax.experimental.pallas import tpu as pltpu
from jax.experimental.pallas import tpu_sc as plsc

N_NODES = 10000
D_FEAT = 128
N_EDGES = 320000

NC = 2
NS = 16
NW = NC * NS

CHUNK = 128
N_CHUNKS = N_EDGES // CHUNK
T_CHUNKS = N_CHUNKS // NW
X_CHUNKS = N_CHUNKS - NW * T_CHUNKS

ROWS_PER_TILE = 632
LAST_ROWS = N_NODES - 15 * ROWS_PER_TILE

NBUF = 3
NIDX = 6


def _sc_partial_sums(x, src, dst, zeros):
  mesh = plsc.VectorSubcoreMesh(core_axis_name="c", subcore_axis_name="s")

  @functools.partial(
      pl.kernel,
      mesh=mesh,
      out_type=jax.ShapeDtypeStruct((NC, N_NODES, D_FEAT), jnp.float32),
      scratch_types=[
          *[pltpu.VMEM((CHUNK,), jnp.int32) for _ in range(NIDX)],
          *[pltpu.VMEM((CHUNK,), jnp.int32) for _ in range(NIDX)],
          *[pltpu.VMEM((CHUNK, D_FEAT), jnp.float32) for _ in range(NBUF)],
          pltpu.VMEM_SHARED((N_NODES, D_FEAT), jnp.float32),
          *[pltpu.SemaphoreType.DMA for _ in range(2 * NBUF + NIDX)],
      ],
  )
  def k(x_hbm, src_hbm, dst_hbm, zeros_hbm, out_hbm, *refs):
    sidx = refs[0:NIDX]
    didx = refs[NIDX:2 * NIDX]
    bufs = refs[2 * NIDX:2 * NIDX + NBUF]
    acc = refs[2 * NIDX + NBUF]
    rest = refs[2 * NIDX + NBUF + 1:]
    gsems = rest[0:NBUF]
    ssems = rest[NBUF:2 * NBUF]
    isems = rest[2 * NBUF:2 * NBUF + NIDX]

    c = lax.axis_index("c")
    s = lax.axis_index("s")
    wid = s * NC + c
    base_chunk = wid * T_CHUNKS
    has_extra = wid < X_CHUNKS
    extra_chunk = NW * T_CHUNKS + wid

    @pl.when(s < 15)
    def _():
      pltpu.sync_copy(zeros_hbm, acc.at[pl.ds(s * ROWS_PER_TILE, ROWS_PER_TILE)])

    @pl.when(s == 15)
    def _():
      pltpu.sync_copy(zeros_hbm.at[pl.ds(0, LAST_ROWS)],
                      acc.at[pl.ds(15 * ROWS_PER_TILE, LAST_ROWS)])

    plsc.subcore_barrier()

    def idx_load(t, j):
      q = j % NIDX
      e = (base_chunk + t) * CHUNK
      return (
          pltpu.make_async_copy(src_hbm.at[pl.ds(e, CHUNK)], sidx[q], isems[q]),
          pltpu.make_async_copy(dst_hbm.at[pl.ds(e, CHUNK)], didx[q], isems[q]),
      )

    def idx_start(t, j):
      a, b = idx_load(t, j)
      a.start()
      b.start()

    def idx_wait(t, j):
      a, b = idx_load(t, j)
      a.wait()
      b.wait()

    def gather(j):
      return pltpu.make_async_copy(
          x_hbm.at[sidx[j % NIDX]], bufs[j % NBUF], gsems[j % NBUF])

    def scatter_start(j):
      pltpu.async_copy(
          bufs[j % NBUF], acc.at[didx[j % NIDX]], ssems[j % NBUF], add=True)

    def scatter_wait(j):
      pltpu.make_async_copy(
          bufs[j % NBUF], acc.at[didx[j % NIDX]], ssems[j % NBUF]).wait()

    for t in range(NIDX - 1):
      idx_start(t, t)
    for t in range(NBUF - 1):
      idx_wait(t, t)
      gather(t).start()

    def chunk_step(t, j):
      @pl.when(t + 2 < T_CHUNKS)
      def _():
        idx_wait(t + 2, j + 2)

      gather(j).wait()
      scatter_start(j)

      @pl.when(t >= 1)
      def _():
        scatter_wait(j - 1)

      @pl.when(t + NIDX - 1 < T_CHUNKS)
      def _():
        idx_start(t + NIDX - 1, j - 1)

      @pl.when(t + 2 < T_CHUNKS)
      def _():
        gather(j + 2).start()

    def body(i, carry):
      t0 = NIDX * i
      for j in range(NIDX):
        chunk_step(t0 + j, j)
      return carry

    lax.fori_loop(0, T_CHUNKS // NIDX, body, 0)
    scatter_wait(T_CHUNKS - 1)

    @pl.when(has_extra)
    def _():
      e = extra_chunk * CHUNK
      pltpu.make_async_copy(src_hbm.at[pl.ds(e, CHUNK)], sidx[0], isems[0]).start()
      pltpu.make_async_copy(dst_hbm.at[pl.ds(e, CHUNK)], didx[0], isems[0]).start()
      pltpu.make_async_copy(src_hbm.at[pl.ds(e, CHUNK)], sidx[0], isems[0]).wait()
      pltpu.make_async_copy(dst_hbm.at[pl.ds(e, CHUNK)], didx[0], isems[0]).wait()
      pltpu.make_async_copy(x_hbm.at[sidx[0]], bufs[0], gsems[0]).start()
      pltpu.make_async_copy(x_hbm.at[sidx[0]], bufs[0], gsems[0]).wait()
      pltpu.sync_copy(bufs[0], acc.at[didx[0]], add=True)

    plsc.subcore_barrier()

    @pl.when(s < 15)
    def _():
      pltpu.sync_copy(
          acc.at[pl.ds(s * ROWS_PER_TILE, ROWS_PER_TILE)],
          out_hbm.at[c, pl.ds(s * ROWS_PER_TILE, ROWS_PER_TILE)],
      )

    @pl.when(s == 15)
    def _():
      pltpu.sync_copy(
          acc.at[pl.ds(15 * ROWS_PER_TILE, LAST_ROWS)],
          out_hbm.at[c, pl.ds(15 * ROWS_PER_TILE, LAST_ROWS)],
      )

  return k(x, src, dst, zeros)


def _combine_body(a_ref, b_ref, o_ref):
  o_ref[...] = a_ref[0] + b_ref[0]


_BLK = 1000


def _combine(partials):
  return pl.pallas_call(
      _combine_body,
      grid=(N_NODES // _BLK,),
      in_specs=[
          pl.BlockSpec((1, _BLK, D_FEAT), lambda i: (0, i, 0)),
          pl.BlockSpec((1, _BLK, D_FEAT), lambda i: (1, i, 0)),
      ],
      out_specs=pl.BlockSpec((_BLK, D_FEAT), lambda i: (i, 0)),
      out_shape=jax.ShapeDtypeStruct((N_NODES, D_FEAT), jnp.float32),
  )(partials, partials)


def kernel(X, edge_index):
  src = edge_index[1]
  dst = edge_index[0]
  zeros = jnp.zeros((ROWS_PER_TILE, D_FEAT), jnp.float32)
  partials = _sc_partial_sums(X, src, dst, zeros)
  return _combine(partials)


# speedup vs baseline: 1.2904x; 1.0565x over previous
import functools

import jax
import jax.numpy as jnp
from jax import lax
from jax.experimental import pallas as pl
from jax.experimental.pallas import tpu as pltpu
from jax.experimental.pallas import tpu_sc as plsc

N_NODES = 10000
D_FEAT = 128
N_EDGES = 320000

NC = 2
NS = 16
NW = NC * NS

CHUNK = 128
N_CHUNKS = N_EDGES // CHUNK
T_CHUNKS = N_CHUNKS // NW
X_CHUNKS = N_CHUNKS - NW * T_CHUNKS

ROWS_PER_TILE = 632
LAST_ROWS = N_NODES - 15 * ROWS_PER_TILE

NBUF = 3
NIDX = 6


def _sc_partial_sums(x, src, dst, zeros):
  mesh = plsc.VectorSubcoreMesh(core_axis_name="c", subcore_axis_name="s")

  @functools.partial(
      pl.kernel,
      mesh=mesh,
      out_type=jax.ShapeDtypeStruct((NC, N_NODES, D_FEAT), jnp.float32),
      scratch_types=[
          *[pltpu.VMEM((CHUNK,), jnp.int32) for _ in range(NIDX)],
          *[pltpu.VMEM((CHUNK,), jnp.int32) for _ in range(NIDX)],
          *[pltpu.VMEM((CHUNK, D_FEAT), jnp.float32) for _ in range(NBUF)],
          pltpu.VMEM_SHARED((N_NODES, D_FEAT), jnp.float32),
          *[pltpu.SemaphoreType.DMA for _ in range(NBUF + NIDX)],
      ],
  )
  def k(x_hbm, src_hbm, dst_hbm, zeros_hbm, out_hbm, *refs):
    sidx = refs[0:NIDX]
    didx = refs[NIDX:2 * NIDX]
    bufs = refs[2 * NIDX:2 * NIDX + NBUF]
    acc = refs[2 * NIDX + NBUF]
    rest = refs[2 * NIDX + NBUF + 1:]
    gsems = rest[0:NBUF]
    isems = rest[NBUF:NBUF + NIDX]

    c = lax.axis_index("c")
    s = lax.axis_index("s")
    wid = s * NC + c
    base_chunk = wid * T_CHUNKS
    has_extra = wid < X_CHUNKS
    extra_chunk = NW * T_CHUNKS + wid

    @pl.when(s < 15)
    def _():
      pltpu.sync_copy(zeros_hbm, acc.at[pl.ds(s * ROWS_PER_TILE, ROWS_PER_TILE)])

    @pl.when(s == 15)
    def _():
      pltpu.sync_copy(zeros_hbm.at[pl.ds(0, LAST_ROWS)],
                      acc.at[pl.ds(15 * ROWS_PER_TILE, LAST_ROWS)])

    plsc.subcore_barrier()

    def idx_load(t, j):
      q = j % NIDX
      e = (base_chunk + t) * CHUNK
      return (
          pltpu.make_async_copy(src_hbm.at[pl.ds(e, CHUNK)], sidx[q], isems[q]),
          pltpu.make_async_copy(dst_hbm.at[pl.ds(e, CHUNK)], didx[q], isems[q]),
      )

    def idx_start(t, j):
      a, b = idx_load(t, j)
      a.start()
      b.start()

    def idx_wait(t, j):
      a, b = idx_load(t, j)
      a.wait()
      b.wait()

    def gather(j):
      return pltpu.make_async_copy(
          x_hbm.at[sidx[j % NIDX]], bufs[j % NBUF], gsems[j % NBUF])

    def scatter_add(j):
      pltpu.sync_copy(bufs[j % NBUF], acc.at[didx[j % NIDX]], add=True)

    for t in range(NIDX - 1):
      idx_start(t, t)
    for t in range(NBUF - 1):
      idx_wait(t, t)
      gather(t).start()

    def chunk_step(t, j):
      idx_wait(t + 2, j + 2)
      gather(j + 2).start()
      gather(j).wait()
      scatter_add(j)
      idx_start(t + 5, j + 5)

    def body(i, carry):
      t0 = NIDX * i
      for j in range(NIDX):
        chunk_step(t0 + j, j)
      return carry

    lax.fori_loop(0, (T_CHUNKS // NIDX) - 1, body, 0)
    for t in range(T_CHUNKS - NIDX, T_CHUNKS):
      j = t % NIDX
      if t + 2 < T_CHUNKS:
        idx_wait(t + 2, j + 2)
        gather(j + 2).start()
      gather(j).wait()
      scatter_add(j)
      if t + 5 < T_CHUNKS:
        idx_start(t + 5, j + 5)

    @pl.when(has_extra)
    def _():
      e = extra_chunk * CHUNK
      pltpu.make_async_copy(src_hbm.at[pl.ds(e, CHUNK)], sidx[0], isems[0]).start()
      pltpu.make_async_copy(dst_hbm.at[pl.ds(e, CHUNK)], didx[0], isems[0]).start()
      pltpu.make_async_copy(src_hbm.at[pl.ds(e, CHUNK)], sidx[0], isems[0]).wait()
      pltpu.make_async_copy(dst_hbm.at[pl.ds(e, CHUNK)], didx[0], isems[0]).wait()
      pltpu.make_async_copy(x_hbm.at[sidx[0]], bufs[0], gsems[0]).start()
      pltpu.make_async_copy(x_hbm.at[sidx[0]], bufs[0], gsems[0]).wait()
      pltpu.sync_copy(bufs[0], acc.at[didx[0]], add=True)

    plsc.subcore_barrier()

    @pl.when(s < 15)
    def _():
      pltpu.sync_copy(
          acc.at[pl.ds(s * ROWS_PER_TILE, ROWS_PER_TILE)],
          out_hbm.at[c, pl.ds(s * ROWS_PER_TILE, ROWS_PER_TILE)],
      )

    @pl.when(s == 15)
    def _():
      pltpu.sync_copy(
          acc.at[pl.ds(15 * ROWS_PER_TILE, LAST_ROWS)],
          out_hbm.at[c, pl.ds(15 * ROWS_PER_TILE, LAST_ROWS)],
      )

  return k(x, src, dst, zeros)


def _combine_body(a_ref, b_ref, o_ref):
  o_ref[...] = a_ref[0] + b_ref[0]


_BLK = 1000


def _combine(partials):
  return pl.pallas_call(
      _combine_body,
      grid=(N_NODES // _BLK,),
      in_specs=[
          pl.BlockSpec((1, _BLK, D_FEAT), lambda i: (0, i, 0)),
          pl.BlockSpec((1, _BLK, D_FEAT), lambda i: (1, i, 0)),
      ],
      out_specs=pl.BlockSpec((_BLK, D_FEAT), lambda i: (i, 0)),
      out_shape=jax.ShapeDtypeStruct((N_NODES, D_FEAT), jnp.float32),
  )(partials, partials)


def kernel(X, edge_index):
  src = edge_index[1]
  dst = edge_index[0]
  zeros = jnp.zeros((ROWS_PER_TILE, D_FEAT), jnp.float32)
  partials = _sc_partial_sums(X, src, dst, zeros)
  return _combine(partials)


# speedup vs baseline: 1.3047x; 1.0110x over previous
import functools

import jax
import jax.numpy as jnp
from jax import lax
from jax.experimental import pallas as pl
from jax.experimental.pallas import tpu as pltpu
from jax.experimental.pallas import tpu_sc as plsc

N_NODES = 10000
D_FEAT = 128
N_EDGES = 320000

NC = 2
NS = 16
NW = NC * NS

CHUNK = 128
N_CHUNKS = N_EDGES // CHUNK
T_CHUNKS = N_CHUNKS // NW
X_CHUNKS = N_CHUNKS - NW * T_CHUNKS

ROWS_PER_TILE = 632
LAST_ROWS = N_NODES - 15 * ROWS_PER_TILE

NBUF = 3
NIDX = 6


def _sc_partial_sums(x, src, dst, zeros):
  mesh = plsc.VectorSubcoreMesh(core_axis_name="c", subcore_axis_name="s")

  @functools.partial(
      pl.kernel,
      mesh=mesh,
      out_type=jax.ShapeDtypeStruct((NC, N_NODES, D_FEAT), jnp.float32),
      scratch_types=[
          *[pltpu.VMEM((CHUNK,), jnp.int32) for _ in range(NIDX)],
          *[pltpu.VMEM((CHUNK,), jnp.int32) for _ in range(NIDX)],
          *[pltpu.VMEM((CHUNK, D_FEAT), jnp.float32) for _ in range(NBUF)],
          pltpu.VMEM_SHARED((N_NODES, D_FEAT), jnp.float32),
          *[pltpu.SemaphoreType.DMA for _ in range(NBUF + NIDX + 1)],
      ],
  )
  def k(x_hbm, src_hbm, dst_hbm, zeros_hbm, out_hbm, *refs):
    sidx = refs[0:NIDX]
    didx = refs[NIDX:2 * NIDX]
    bufs = refs[2 * NIDX:2 * NIDX + NBUF]
    acc = refs[2 * NIDX + NBUF]
    rest = refs[2 * NIDX + NBUF + 1:]
    gsems = rest[0:NBUF]
    isems = rest[NBUF:NBUF + NIDX]
    zsem = rest[NBUF + NIDX]

    c = lax.axis_index("c")
    s = lax.axis_index("s")
    wid = s * NC + c
    base_chunk = wid * T_CHUNKS
    has_extra = wid < X_CHUNKS
    extra_chunk = NW * T_CHUNKS + wid

    def zero_copy():
      a = pltpu.make_async_copy(
          zeros_hbm, acc.at[pl.ds(s * ROWS_PER_TILE, ROWS_PER_TILE)], zsem)
      b = pltpu.make_async_copy(
          zeros_hbm.at[pl.ds(0, LAST_ROWS)],
          acc.at[pl.ds(15 * ROWS_PER_TILE, LAST_ROWS)], zsem)
      return a, b

    @pl.when(s < 15)
    def _():
      zero_copy()[0].start()

    @pl.when(s == 15)
    def _():
      zero_copy()[1].start()

    def idx_load(t, j):
      q = j % NIDX
      e = (base_chunk + t) * CHUNK
      return (
          pltpu.make_async_copy(src_hbm.at[pl.ds(e, CHUNK)], sidx[q], isems[q]),
          pltpu.make_async_copy(dst_hbm.at[pl.ds(e, CHUNK)], didx[q], isems[q]),
      )

    def idx_start(t, j):
      a, b = idx_load(t, j)
      a.start()
      b.start()

    def idx_wait(t, j):
      a, b = idx_load(t, j)
      a.wait()
      b.wait()

    def gather(j):
      return pltpu.make_async_copy(
          x_hbm.at[sidx[j % NIDX]], bufs[j % NBUF], gsems[j % NBUF])

    def scatter_add(j):
      pltpu.sync_copy(bufs[j % NBUF], acc.at[didx[j % NIDX]], add=True)

    for t in range(NIDX - 1):
      idx_start(t, t)
    for t in range(NBUF - 1):
      idx_wait(t, t)
      gather(t).start()

    @pl.when(s < 15)
    def _():
      zero_copy()[0].wait()

    @pl.when(s == 15)
    def _():
      zero_copy()[1].wait()

    plsc.subcore_barrier()

    def chunk_step(t, j):
      idx_wait(t + 2, j + 2)
      gather(j + 2).start()
      gather(j).wait()
      scatter_add(j)
      idx_start(t + 5, j + 5)

    def body(i, carry):
      t0 = NIDX * i
      for j in range(NIDX):
        chunk_step(t0 + j, j)
      return carry

    lax.fori_loop(0, (T_CHUNKS // NIDX) - 1, body, 0)
    for t in range(T_CHUNKS - NIDX, T_CHUNKS):
      j = t % NIDX
      if t + 2 < T_CHUNKS:
        idx_wait(t + 2, j + 2)
        gather(j + 2).start()
      gather(j).wait()
      scatter_add(j)
      if t + 5 < T_CHUNKS:
        idx_start(t + 5, j + 5)

    @pl.when(has_extra)
    def _():
      e = extra_chunk * CHUNK
      pltpu.make_async_copy(src_hbm.at[pl.ds(e, CHUNK)], sidx[0], isems[0]).start()
      pltpu.make_async_copy(dst_hbm.at[pl.ds(e, CHUNK)], didx[0], isems[0]).start()
      pltpu.make_async_copy(src_hbm.at[pl.ds(e, CHUNK)], sidx[0], isems[0]).wait()
      pltpu.make_async_copy(dst_hbm.at[pl.ds(e, CHUNK)], didx[0], isems[0]).wait()
      pltpu.make_async_copy(x_hbm.at[sidx[0]], bufs[0], gsems[0]).start()
      pltpu.make_async_copy(x_hbm.at[sidx[0]], bufs[0], gsems[0]).wait()
      pltpu.sync_copy(bufs[0], acc.at[didx[0]], add=True)

    plsc.subcore_barrier()

    @pl.when(s < 15)
    def _():
      pltpu.sync_copy(
          acc.at[pl.ds(s * ROWS_PER_TILE, ROWS_PER_TILE)],
          out_hbm.at[c, pl.ds(s * ROWS_PER_TILE, ROWS_PER_TILE)],
      )

    @pl.when(s == 15)
    def _():
      pltpu.sync_copy(
          acc.at[pl.ds(15 * ROWS_PER_TILE, LAST_ROWS)],
          out_hbm.at[c, pl.ds(15 * ROWS_PER_TILE, LAST_ROWS)],
      )

  return k(x, src, dst, zeros)


def _combine_body(a_ref, b_ref, o_ref):
  o_ref[...] = a_ref[0] + b_ref[0]


_BLK = 1000


def _combine(partials):
  return pl.pallas_call(
      _combine_body,
      grid=(N_NODES // _BLK,),
      in_specs=[
          pl.BlockSpec((1, _BLK, D_FEAT), lambda i: (0, i, 0)),
          pl.BlockSpec((1, _BLK, D_FEAT), lambda i: (1, i, 0)),
      ],
      out_specs=pl.BlockSpec((_BLK, D_FEAT), lambda i: (i, 0)),
      out_shape=jax.ShapeDtypeStruct((N_NODES, D_FEAT), jnp.float32),
  )(partials, partials)


def kernel(X, edge_index):
  src = edge_index[1]
  dst = edge_index[0]
  zeros = jnp.zeros((ROWS_PER_TILE, D_FEAT), jnp.float32)
  partials = _sc_partial_sums(X, src, dst, zeros)
  return _combine(partials)


# speedup vs baseline: 1.3245x; 1.0152x over previous
import functools

import jax
import jax.numpy as jnp
from jax import lax
from jax.experimental import pallas as pl
from jax.experimental.pallas import tpu as pltpu
from jax.experimental.pallas import tpu_sc as plsc

N_NODES = 10000
D_FEAT = 128
N_EDGES = 320000

NC = 2
NS = 16
NW = NC * NS

CHUNK = 128
N_CHUNKS = N_EDGES // CHUNK
T_CHUNKS = N_CHUNKS // NW
X_CHUNKS = N_CHUNKS - NW * T_CHUNKS

ROWS_PER_TILE = 632
LAST_ROWS = N_NODES - 15 * ROWS_PER_TILE

NBUF = 3
NIDX = 6


def _sc_partial_sums(x, src, dst, zeros):
  mesh = plsc.VectorSubcoreMesh(core_axis_name="c", subcore_axis_name="s")

  @functools.partial(
      pl.kernel,
      mesh=mesh,
      out_type=jax.ShapeDtypeStruct((NC, N_NODES, D_FEAT), jnp.float32),
      scratch_types=[
          *[pltpu.VMEM((CHUNK,), jnp.int32) for _ in range(NIDX)],
          *[pltpu.VMEM((CHUNK,), jnp.int32) for _ in range(NIDX)],
          *[pltpu.VMEM((CHUNK, D_FEAT), jnp.float32) for _ in range(NBUF)],
          pltpu.VMEM_SHARED((N_NODES, D_FEAT), jnp.float32),
          *[pltpu.SemaphoreType.DMA for _ in range(NBUF + NIDX + 1)],
      ],
  )
  def k(x_hbm, src_hbm, dst_hbm, zeros_hbm, out_hbm, *refs):
    sidx = refs[0:NIDX]
    didx = refs[NIDX:2 * NIDX]
    bufs = refs[2 * NIDX:2 * NIDX + NBUF]
    acc = refs[2 * NIDX + NBUF]
    rest = refs[2 * NIDX + NBUF + 1:]
    gsems = rest[0:NBUF]
    isems = rest[NBUF:NBUF + NIDX]
    zsem = rest[NBUF + NIDX]

    c = lax.axis_index("c")
    s = lax.axis_index("s")
    wid = s * NC + c
    base_chunk = wid * T_CHUNKS
    has_extra = wid < X_CHUNKS
    extra_chunk = NW * T_CHUNKS + wid

    def zero_copy():
      a = pltpu.make_async_copy(
          zeros_hbm, acc.at[pl.ds(s * ROWS_PER_TILE, ROWS_PER_TILE)], zsem)
      b = pltpu.make_async_copy(
          zeros_hbm.at[pl.ds(0, LAST_ROWS)],
          acc.at[pl.ds(15 * ROWS_PER_TILE, LAST_ROWS)], zsem)
      return a, b

    @pl.when(s < 15)
    def _():
      zero_copy()[0].start()

    @pl.when(s == 15)
    def _():
      zero_copy()[1].start()

    def idx_load(t, j):
      q = j % NIDX
      e = (base_chunk + t) * CHUNK
      return (
          pltpu.make_async_copy(src_hbm.at[pl.ds(e, CHUNK)], sidx[q], isems[q]),
          pltpu.make_async_copy(dst_hbm.at[pl.ds(e, CHUNK)], didx[q], isems[q]),
      )

    def idx_start(t, j):
      a, b = idx_load(t, j)
      a.start()
      b.start()

    def idx_wait(t, j):
      a, b = idx_load(t, j)
      a.wait()
      b.wait()

    def gather(j):
      return pltpu.make_async_copy(
          x_hbm.at[sidx[j % NIDX]], bufs[j % NBUF], gsems[j % NBUF])

    def scatter_add(j):
      pltpu.sync_copy(bufs[j % NBUF], acc.at[didx[j % NIDX]], add=True)

    for t in range(NIDX - 1):
      idx_start(t, t)
    for t in range(NBUF - 1):
      idx_wait(t, t)
      gather(t).start()

    @pl.when(s < 15)
    def _():
      zero_copy()[0].wait()

    @pl.when(s == 15)
    def _():
      zero_copy()[1].wait()

    plsc.subcore_barrier()

    def chunk_step(t, j):
      idx_wait(t + 2, j + 2)
      gather(j + 2).start()
      gather(j).wait()
      scatter_add(j)
      idx_start(t + 5, j + 5)

    def body(i, carry):
      t0 = NIDX * i
      for j in range(NIDX):
        chunk_step(t0 + j, j)
      return carry

    def extra_idx(e):
      return (
          pltpu.make_async_copy(src_hbm.at[pl.ds(e, CHUNK)], sidx[0], isems[0]),
          pltpu.make_async_copy(dst_hbm.at[pl.ds(e, CHUNK)], didx[0], isems[0]),
      )

    def extra_gather():
      return pltpu.make_async_copy(x_hbm.at[sidx[0]], bufs[0], gsems[0])

    lax.fori_loop(0, (T_CHUNKS // NIDX) - 1, body, 0)
    for t in range(T_CHUNKS - NIDX, T_CHUNKS):
      j = t % NIDX
      if t == T_CHUNKS - 5:
        @pl.when(has_extra)
        def _():
          a, b = extra_idx(extra_chunk * CHUNK)
          a.start()
          b.start()
      if t == T_CHUNKS - 2:
        @pl.when(has_extra)
        def _():
          a, b = extra_idx(extra_chunk * CHUNK)
          a.wait()
          b.wait()
          extra_gather().start()
      if t + 2 < T_CHUNKS:
        idx_wait(t + 2, j + 2)
        gather(j + 2).start()
      gather(j).wait()
      scatter_add(j)
      if t + 5 < T_CHUNKS:
        idx_start(t + 5, j + 5)

    @pl.when(has_extra)
    def _():
      extra_gather().wait()
      pltpu.sync_copy(bufs[0], acc.at[didx[0]], add=True)

    plsc.subcore_barrier()

    @pl.when(s < 15)
    def _():
      pltpu.sync_copy(
          acc.at[pl.ds(s * ROWS_PER_TILE, ROWS_PER_TILE)],
          out_hbm.at[c, pl.ds(s * ROWS_PER_TILE, ROWS_PER_TILE)],
      )

    @pl.when(s == 15)
    def _():
      pltpu.sync_copy(
          acc.at[pl.ds(15 * ROWS_PER_TILE, LAST_ROWS)],
          out_hbm.at[c, pl.ds(15 * ROWS_PER_TILE, LAST_ROWS)],
      )

  return k(x, src, dst, zeros)


def _combine_body(a_ref, b_ref, o_ref):
  o_ref[...] = a_ref[0] + b_ref[0]


_BLK = 1000


def _combine(partials):
  return pl.pallas_call(
      _combine_body,
      grid=(N_NODES // _BLK,),
      in_specs=[
          pl.BlockSpec((1, _BLK, D_FEAT), lambda i: (0, i, 0)),
          pl.BlockSpec((1, _BLK, D_FEAT), lambda i: (1, i, 0)),
      ],
      out_specs=pl.BlockSpec((_BLK, D_FEAT), lambda i: (i, 0)),
      out_shape=jax.ShapeDtypeStruct((N_NODES, D_FEAT), jnp.float32),
  )(partials, partials)


def kernel(X, edge_index):
  src = edge_index[1]
  dst = edge_index[0]
  zeros = jnp.zeros((ROWS_PER_TILE, D_FEAT), jnp.float32)
  partials = _sc_partial_sums(X, src, dst, zeros)
  return _combine(partials)
